# baseline (speedup 1.0000x reference)
import functools

import jax
import jax.numpy as jnp
from jax.experimental import pallas as pl
from jax.experimental.pallas import tpu as pltpu


def _body(x_ref, o_ref, *, inv_hw):
    o_ref[...] = jnp.sum(x_ref[...], axis=1, keepdims=True,
                         dtype=jnp.float32) * inv_hw


def kernel(x):
    N, C, H, W = x.shape
    hw = H * W
    inv_hw = 1.0 / float(hw)

    xt = jnp.transpose(x.reshape(N, C, hw), (0, 2, 1))

    out = pl.pallas_call(
        functools.partial(_body, inv_hw=inv_hw),
        out_shape=jax.ShapeDtypeStruct((N, 1, C), jnp.float32),
        grid=(N,),
        in_specs=[pl.BlockSpec((1, hw, C), lambda i: (i, 0, 0))],
        out_specs=pl.BlockSpec((1, 1, C), lambda i: (i, 0, 0)),
        compiler_params=pltpu.CompilerParams(
            dimension_semantics=("parallel",),
            vmem_limit_bytes=64 << 20,
        ),
        cost_estimate=pl.CostEstimate(
            flops=N * C * hw, transcendentals=0,
            bytes_accessed=N * C * hw * 4 + N * C * 4),
    )(xt)

    return out.reshape(N, C, 1, 1)
